# baseline (speedup 1.0000x reference)
import functools

import jax
import jax.numpy as jnp
from jax import lax
from jax.experimental import pallas as pl
from jax.experimental.pallas import tpu as pltpu
from jax.experimental.pallas import tpu_sc as plsc

_CHUNK = 128


def _build_sc_gather(B, dims):
    info = plsc.get_sparse_core_info()
    NC, NS = info.num_cores, info.num_subcores
    NW = NC * NS
    bpw = B // NW
    nch = bpw // _CHUNK
    f32 = jnp.float32
    mesh = plsc.VectorSubcoreMesh(core_axis_name="c", subcore_axis_name="s")

    @functools.partial(
        pl.kernel,
        out_type=tuple(jax.ShapeDtypeStruct((B, d), f32) for d in dims + dims),
        mesh=mesh,
        scratch_types=[
            pltpu.VMEM((nch, _CHUNK), jnp.int32),
            pltpu.VMEM((nch, _CHUNK), jnp.int32),
        ] + [pltpu.VMEM((bpw, d), f32) for d in dims + dims] + [
            pltpu.SemaphoreType.DMA,
        ],
        compiler_params=pltpu.CompilerParams(use_tc_tiling_on_sc=False),
    )
    def sc_gather(u2_hbm, i2_hbm, tug, tut, tud, tig, tit, tid,
                  oug, out_, oud, oig, oit, oid,
                  idx_u, idx_i, rug, rut, rud, rig, rit, rid, sem):
        wid = lax.axis_index("s") * NC + lax.axis_index("c")
        base = wid * bpw
        pltpu.sync_copy(u2_hbm.at[pl.ds(wid * nch, nch)], idx_u)
        pltpu.sync_copy(i2_hbm.at[pl.ds(wid * nch, nch)], idx_i)
        copies = []
        for tab, buf, idx in ((tug, rug, idx_u), (tut, rut, idx_u),
                              (tud, rud, idx_u), (tig, rig, idx_i),
                              (tit, rit, idx_i), (tid, rid, idx_i)):
            for j in range(nch):
                copies.append(pltpu.async_copy(
                    tab.at[idx.at[j]], buf.at[pl.ds(j * _CHUNK, _CHUNK)], sem))
        for c in copies:
            c.wait()
        for buf, out in ((rug, oug), (rut, out_), (rud, oud),
                         (rig, oig), (rit, oit), (rid, oid)):
            pltpu.sync_copy(buf, out.at[pl.ds(base, bpw)])

    return sc_gather


def _dense_body(ug, ig, ut, itt, ud, idd, au, ai, atu, ati, adu, adi,
                b6, wo, out):
    hp = jax.lax.Precision.HIGHEST
    pre = (jnp.dot(ug[...], au[...], precision=hp, preferred_element_type=jnp.float32)
           + jnp.dot(ig[...], ai[...], precision=hp, preferred_element_type=jnp.float32)
           + jnp.dot(ut[...], atu[...], precision=hp, preferred_element_type=jnp.float32)
           + jnp.dot(itt[...], ati[...], precision=hp, preferred_element_type=jnp.float32)
           + jnp.dot(ud[...], adu[...], precision=hp, preferred_element_type=jnp.float32)
           + jnp.dot(idd[...], adi[...], precision=hp, preferred_element_type=jnp.float32)
           + b6[...])
    x = jnp.maximum(pre, 0.0)
    out[...] = jnp.sum(x * wo[...], axis=1, keepdims=True)


def _dense(B, feats, mats, b6, wo):
    BLK = 2048
    full = lambda shape: pl.BlockSpec(shape, lambda i: (0, 0))
    return pl.pallas_call(
        _dense_body,
        grid=(B // BLK,),
        in_specs=[pl.BlockSpec((BLK, f.shape[1]), lambda i: (i, 0)) for f in feats]
        + [full(m.shape) for m in mats] + [full((1, 8)), full((1, 8))],
        out_specs=pl.BlockSpec((BLK, 1), lambda i: (i, 0)),
        out_shape=jax.ShapeDtypeStruct((B, 1), jnp.float32),
    )(*feats, *mats, b6, wo)


def kernel(user, item, user_genres, user_titles, user_dirs,
           item_genres, item_titles, item_dirs,
           W_g, b_g, W_t, b_t, W_d, b_d, W_out, b_out):
    B = user.shape[0]
    dims = (user_genres.shape[1], user_titles.shape[1], user_dirs.shape[1])
    u2 = user.astype(jnp.int32).reshape(B // _CHUNK, _CHUNK)
    i2 = item.astype(jnp.int32).reshape(B // _CHUNK, _CHUNK)

    feats = _build_sc_gather(B, dims)(
        u2, i2, user_genres, user_titles, user_dirs,
        item_genres, item_titles, item_dirs)

    mats = []
    for col, (W, d) in enumerate(((W_g, dims[0]), (W_t, dims[1]), (W_d, dims[2]))):
        for half in (0, 1):
            m = jnp.zeros((d, 8), jnp.float32)
            mats.append(m.at[:, 2 * col:2 * col + 2].set(W[:, half * d:(half + 1) * d].T))
    b6 = jnp.concatenate([b_g, b_t, b_d, jnp.ones((1,), jnp.float32),
                          jnp.zeros((1,), jnp.float32)]).reshape(1, 8)
    wo = jnp.concatenate([W_out[0], b_out, jnp.zeros((1,), jnp.float32)]).reshape(1, 8)

    ordered = (feats[0], feats[3], feats[1], feats[4], feats[2], feats[5])
    return _dense(B, ordered, mats, b6, wo)
